# baseline (speedup 1.0000x reference)
import functools

import jax
import jax.numpy as jnp
import numpy as np
from jax import lax
from jax.experimental import pallas as pl
from jax.experimental.pallas import tpu as pltpu
from jax.experimental.pallas import tpu_sc as plsc

T = 2048
H = 1024
NH = 16
NKV = 8
HD = 64
E = 8
TOPK = 2
FF = 512
EPS = 1e-6
THETA = 10000.0

BT = 256
BQ = 256
TILE = 128
PADMAX = TOPK * T + E * TILE
NT = PADMAX // TILE
NEG = -1e30

SC_NC = 2
SC_NS = 16
SC_NW = SC_NC * SC_NS


NQK = (NH + NKV) * HD

_G_np = np.zeros((NQK, NH + NKV), dtype=np.float32)
for _c in range(NQK):
    _G_np[_c, _c // HD] = 1.0


def _qkv_body(x_ref, cosb_ref, sinb_ref, wqkv_ref, ln1_ref, wtile_ref,
              g_ref, gt_ref, q_ref, k_ref, v_ref):
    x = x_ref[...]
    var = jnp.mean(x * x, axis=-1, keepdims=True)
    xn = x * lax.rsqrt(var + EPS) * ln1_ref[...]
    qkv = jnp.dot(xn, wqkv_ref[...], preferred_element_type=jnp.float32)
    qk = qkv[:, :NQK]
    ss = jnp.dot(qk * qk, g_ref[...], preferred_element_type=jnp.float32)
    rn = lax.rsqrt(ss * (1.0 / HD) + EPS)
    rnb = jnp.dot(rn, gt_ref[...], preferred_element_type=jnp.float32)
    qkn = qk * rnb * wtile_ref[...]
    rm = jnp.roll(qkn, -HD // 2, axis=1)
    rp = jnp.roll(qkn, HD // 2, axis=1)
    col = lax.broadcasted_iota(jnp.int32, (BT, NQK), 1)
    first_half = (col % HD) < (HD // 2)
    xswap = jnp.where(first_half, -rm, rp)
    rot = qkn * cosb_ref[...] + xswap * sinb_ref[...]
    q_ref[...] = rot[:, :NH * HD]
    k_ref[...] = rot[:, NH * HD:]
    v_ref[...] = qkv[:, NQK:]


def _qkv_call(h, cosb, sinb, Wqkv, ln1_w, wtile, G, GT):
    nb = T // BT
    return pl.pallas_call(
        _qkv_body,
        grid=(nb,),
        in_specs=[
            pl.BlockSpec((BT, H), lambda i: (i, 0)),
            pl.BlockSpec((BT, NQK), lambda i: (i, 0)),
            pl.BlockSpec((BT, NQK), lambda i: (i, 0)),
            pl.BlockSpec((H, (NH + 2 * NKV) * HD), lambda i: (0, 0)),
            pl.BlockSpec((1, H), lambda i: (0, 0)),
            pl.BlockSpec((1, NQK), lambda i: (0, 0)),
            pl.BlockSpec((NQK, NH + NKV), lambda i: (0, 0)),
            pl.BlockSpec((NH + NKV, NQK), lambda i: (0, 0)),
        ],
        out_specs=[
            pl.BlockSpec((BT, NH * HD), lambda i: (i, 0)),
            pl.BlockSpec((BT, NKV * HD), lambda i: (i, 0)),
            pl.BlockSpec((BT, NKV * HD), lambda i: (i, 0)),
        ],
        out_shape=[
            jax.ShapeDtypeStruct((T, NH * HD), jnp.float32),
            jax.ShapeDtypeStruct((T, NKV * HD), jnp.float32),
            jax.ShapeDtypeStruct((T, NKV * HD), jnp.float32),
        ],
        compiler_params=pltpu.CompilerParams(
            dimension_semantics=("arbitrary",)),
    )(h, cosb, sinb, Wqkv, ln1_w, wtile, G, GT)


def _attn_body(q_ref, k_ref, v_ref, o_ref, *, qb0, kvlen):
    i = pl.program_id(0)
    rep = NH // NKV
    scale = HD ** -0.5
    row = (i + qb0) * BQ + lax.broadcasted_iota(jnp.int32, (BQ, kvlen), 0)
    col = lax.broadcasted_iota(jnp.int32, (BQ, kvlen), 1)
    causal = col <= row
    os = []
    for h in range(NH):
        kh = h // rep
        q = q_ref[:, h * HD:(h + 1) * HD] * scale
        k = k_ref[:, kh * HD:(kh + 1) * HD]
        s = lax.dot_general(q, k, (((1,), (1,)), ((), ())),
                            preferred_element_type=jnp.float32)
        s = jnp.where(causal, s, NEG)
        m = jnp.max(s, axis=-1, keepdims=True)
        p = jnp.exp(s - m)
        l = jnp.sum(p, axis=-1, keepdims=True)
        v = v_ref[:, kh * HD:(kh + 1) * HD]
        os.append(jnp.dot(p, v, preferred_element_type=jnp.float32) / l)
    o_ref[...] = jnp.concatenate(os, axis=-1)


def _attn_call_part(q, k, v, qb0, nqb, kvlen):
    return pl.pallas_call(
        functools.partial(_attn_body, qb0=qb0, kvlen=kvlen),
        grid=(nqb,),
        in_specs=[
            pl.BlockSpec((BQ, NH * HD), lambda i: (i + qb0, 0)),
            pl.BlockSpec((kvlen, NKV * HD), lambda i: (0, 0)),
            pl.BlockSpec((kvlen, NKV * HD), lambda i: (0, 0)),
        ],
        out_specs=pl.BlockSpec((BQ, NH * HD), lambda i: (i, 0)),
        out_shape=jax.ShapeDtypeStruct((nqb * BQ, NH * HD), jnp.float32),
        compiler_params=pltpu.CompilerParams(
            dimension_semantics=("arbitrary",)),
    )(q, k, v)


def _oproj_body(a0_ref, a1_ref, res_ref, wo_ref, ln2_ref,
                wg_ref, hs2_ref, h2_ref, wfull_ref):
    i = pl.program_id(0)
    attn = jnp.where(i < (T // BT) // 2, a0_ref[...], a1_ref[...])
    a = jnp.dot(attn, wo_ref[...],
                preferred_element_type=jnp.float32) + res_ref[...]
    hs2_ref[...] = a
    var = jnp.mean(a * a, axis=-1, keepdims=True)
    h2 = a * lax.rsqrt(var + EPS) * ln2_ref[...]
    h2_ref[...] = h2
    logits = jnp.dot(h2, wg_ref[...], preferred_element_type=jnp.float32)
    m = jnp.max(logits, axis=-1, keepdims=True)
    p = jnp.exp(logits - m)
    p = p / jnp.sum(p, axis=-1, keepdims=True)
    ie = lax.broadcasted_iota(jnp.int32, (BT, E), 1)
    m1 = jnp.max(p, axis=-1, keepdims=True)
    i1 = jnp.min(jnp.where(p == m1, ie, E), axis=-1, keepdims=True)
    p2 = jnp.where(ie == i1, NEG, p)
    m2 = jnp.max(p2, axis=-1, keepdims=True)
    i2 = jnp.min(jnp.where(p2 == m2, ie, E), axis=-1, keepdims=True)
    denom = m1 + m2
    wfull_ref[...] = (jnp.where(ie == i1, m1 / denom, 0.0)
                      + jnp.where(ie == i2, m2 / denom, 0.0))


def _oproj_call(attn_parts, res, Wo, ln2_w, Wg):
    nb = T // BT
    return pl.pallas_call(
        _oproj_body,
        grid=(nb,),
        in_specs=[
            pl.BlockSpec((BT, NH * HD),
                         lambda i, p=p: (jnp.clip(i - 4 * p, 0, 3), 0))
            for p in range(2)
        ] + [
            pl.BlockSpec((BT, H), lambda i: (i, 0)),
            pl.BlockSpec((NH * HD, H), lambda i: (0, 0)),
            pl.BlockSpec((1, H), lambda i: (0, 0)),
            pl.BlockSpec((H, E), lambda i: (0, 0)),
        ],
        out_specs=[
            pl.BlockSpec((BT, H), lambda i: (i, 0)),
            pl.BlockSpec((BT, H), lambda i: (i, 0)),
            pl.BlockSpec((BT, E), lambda i: (i, 0)),
        ],
        out_shape=[
            jax.ShapeDtypeStruct((T, H), jnp.float32),
            jax.ShapeDtypeStruct((T, H), jnp.float32),
            jax.ShapeDtypeStruct((T, E), jnp.float32),
        ],
        compiler_params=pltpu.CompilerParams(
            dimension_semantics=("arbitrary",)),
    )(*attn_parts, res, Wo, ln2_w, Wg)


@functools.lru_cache(maxsize=None)
def _make_sc_gather(n_rows, width):
    n_per = n_rows // SC_NW
    ch = min(32, n_per)
    nch = n_per // ch
    nbuf = min(3, nch)
    mesh = plsc.VectorSubcoreMesh(core_axis_name="c", subcore_axis_name="s")

    @functools.partial(
        pl.kernel, mesh=mesh,
        out_type=jax.ShapeDtypeStruct((n_rows, width), jnp.int32),
        scratch_types=[
            pltpu.VMEM((n_per,), jnp.int32),
        ] + [pltpu.VMEM((ch, width), jnp.int32)] * nbuf
          + [pltpu.SemaphoreType.DMA] * (2 * nbuf),
    )
    def gk(table_hbm, idx_hbm, out_hbm, idx_v, *scr):
        bufs = scr[:nbuf]
        gsems = scr[nbuf:2 * nbuf]
        osems = scr[2 * nbuf:]
        wid = lax.axis_index("s") * SC_NC + lax.axis_index("c")
        base = wid * n_per
        pltpu.sync_copy(idx_hbm.at[pl.ds(base, n_per)], idx_v)

        def start_gather(c):
            b = c % nbuf
            return pltpu.async_copy(
                table_hbm.at[idx_v.at[pl.ds(c * ch, ch)]], bufs[b], gsems[b])

        g = [None] * nch
        o = [None] * nch
        for c in range(nbuf):
            g[c] = start_gather(c)
        for c in range(nch):
            g[c].wait()
            o[c] = pltpu.async_copy(
                bufs[c % nbuf], out_hbm.at[pl.ds(base + c * ch, ch)],
                osems[c % nbuf])
            if c + nbuf < nch:
                o[c].wait()
                g[c + nbuf] = start_gather(c + nbuf)
        for c in range(max(0, nch - nbuf), nch):
            o[c].wait()

    return gk


@functools.lru_cache(maxsize=None)
def _make_sc_dispatch():
    n_per = T // SC_NW
    ch = min(32, n_per)
    nch = n_per // ch
    mesh = plsc.VectorSubcoreMesh(core_axis_name="c", subcore_axis_name="s")

    @functools.partial(
        pl.kernel, mesh=mesh,
        out_type=jax.ShapeDtypeStruct((PADMAX, H), jnp.float32),
        scratch_types=[
            pltpu.VMEM((n_per,), jnp.int32),
            pltpu.VMEM((n_per,), jnp.int32),
        ] + [pltpu.VMEM((ch, H), jnp.float32)] * nch
          + [pltpu.SemaphoreType.DMA] * (3 * nch),
    )
    def sk(src_hbm, idx_hbm, out_hbm, idx0_v, idx1_v, *scr):
        bufs = scr[:nch]
        lsems = scr[nch:2 * nch]
        ssems = scr[2 * nch:]
        wid = lax.axis_index("s") * SC_NC + lax.axis_index("c")
        base = wid * n_per
        pltpu.sync_copy(idx_hbm.at[pl.ds(base, n_per)], idx0_v)
        pltpu.sync_copy(idx_hbm.at[pl.ds(T + base, n_per)], idx1_v)
        loads = [pltpu.async_copy(src_hbm.at[pl.ds(base + c * ch, ch)],
                                  bufs[c], lsems[c]) for c in range(nch)]
        outs = []
        for c in range(nch):
            loads[c].wait()
            outs.append(pltpu.async_copy(
                bufs[c], out_hbm.at[idx0_v.at[pl.ds(c * ch, ch)]],
                ssems[2 * c]))
            outs.append(pltpu.async_copy(
                bufs[c], out_hbm.at[idx1_v.at[pl.ds(c * ch, ch)]],
                ssems[2 * c + 1]))
        for o in outs:
            o.wait()

    return sk


def _sc_gather_rows(table, idx):
    dt = table.dtype
    if dt.itemsize == 2:
        ti = lax.bitcast_convert_type(
            table.reshape(table.shape[0], -1, 2), jnp.int32)
    else:
        ti = lax.bitcast_convert_type(table, jnp.int32)
    out = _make_sc_gather(idx.shape[0], ti.shape[1])(ti, idx)
    res = lax.bitcast_convert_type(out, dt)
    return res.reshape(idx.shape[0], H)


def _moe_body(te_ref, act_ref, x_ref, wgu_ref, wd_ref, y_ref):
    g_id = pl.program_id(0)

    @pl.when(act_ref[g_id] > 0)
    def _():
        x = x_ref[...]
        gu = jnp.dot(x, wgu_ref[0], preferred_element_type=jnp.float32)
        g = gu[:, :FF]
        u = gu[:, FF:]
        act = g * jax.nn.sigmoid(g) * u
        y_ref[...] = jnp.dot(act, wd_ref[0],
                             preferred_element_type=jnp.float32)


def _moe_call(x_sorted, tile_expert, tile_active, W_gateup, W_down):
    grid_spec = pltpu.PrefetchScalarGridSpec(
        num_scalar_prefetch=2,
        grid=(NT,),
        in_specs=[
            pl.BlockSpec((TILE, H), lambda g, te, ta: (g, 0)),
            pl.BlockSpec((1, H, 2 * FF), lambda g, te, ta: (te[g], 0, 0)),
            pl.BlockSpec((1, FF, H), lambda g, te, ta: (te[g], 0, 0)),
        ],
        out_specs=pl.BlockSpec((TILE, H), lambda g, te, ta: (g, 0)),
    )
    return pl.pallas_call(
        _moe_body,
        grid_spec=grid_spec,
        out_shape=jax.ShapeDtypeStruct((PADMAX, H), jnp.float32),
        compiler_params=pltpu.CompilerParams(
            dimension_semantics=("arbitrary",)),
    )(tile_expert, tile_active, x_sorted, W_gateup, W_down)


def _combine_body(res_ref, g0_ref, g1_ref, w0_ref, w1_ref, out_ref):
    out_ref[...] = (res_ref[...]
                    + g0_ref[...].astype(jnp.float32) * w0_ref[:, :1]
                    + g1_ref[...].astype(jnp.float32) * w1_ref[:, :1])


def _combine_call(res, g01, w0b, w1b):
    nb = T // BT
    return pl.pallas_call(
        _combine_body,
        grid=(nb,),
        in_specs=[
            pl.BlockSpec((BT, H), lambda i: (i, 0)),
            pl.BlockSpec((BT, H), lambda i: (i, 0)),
            pl.BlockSpec((BT, H), lambda i: (i + T // BT, 0)),
            pl.BlockSpec((BT, 128), lambda i: (i, 0)),
            pl.BlockSpec((BT, 128), lambda i: (i, 0)),
        ],
        out_specs=pl.BlockSpec((BT, H), lambda i: (i, 0)),
        out_shape=jax.ShapeDtypeStruct((T, H), jnp.float32),
        compiler_params=pltpu.CompilerParams(
            dimension_semantics=("arbitrary",)),
    )(res, g01, g01, w0b, w1b)


def kernel(hidden_states, positions, Wqkv, Wo, q_norm_w, k_norm_w,
           ln1_w, ln2_w, Wg, W_gateup, W_down):
    f32 = jnp.float32
    inv_freq = 1.0 / (THETA ** (np.arange(0, HD, 2, dtype=np.float32) / HD))
    freqs = positions.astype(f32)[:, None] * inv_freq[None, :]
    cos64 = jnp.cos(jnp.concatenate([freqs, freqs], axis=1))
    sin64 = jnp.sin(jnp.concatenate([freqs, freqs], axis=1))
    cosb = jnp.tile(cos64, (1, NH + NKV))
    sinb = jnp.tile(sin64, (1, NH + NKV))
    wtile = jnp.concatenate(
        [jnp.tile(q_norm_w, NH), jnp.tile(k_norm_w, NKV)]).reshape(1, NQK)
    G = jnp.asarray(_G_np)

    q, k, v = _qkv_call(hidden_states, cosb, sinb, Wqkv,
                        ln1_w.reshape(1, H), wtile, G, G.T)
    attn_parts = [
        _attn_call_part(q, k, v, 4 * p, 4, (p + 1) * (T // 2))
        for p in range(2)
    ]
    hs2, h2, wfull = _oproj_call(attn_parts, hidden_states, Wo,
                                 ln2_w.reshape(1, H), Wg)

    cnt = (wfull > 0.0).astype(jnp.int32)
    csum = jnp.cumsum(cnt, axis=0)
    prefix = csum - cnt
    counts = csum[-1]
    pcounts = ((counts + TILE - 1) // TILE) * TILE
    pend = jnp.cumsum(pcounts)
    poff = pend - pcounts
    pos = poff[None, :] + prefix
    tile_start = jnp.arange(NT, dtype=jnp.int32) * TILE
    tile_expert = jnp.minimum(
        jnp.searchsorted(pend, tile_start, side="right"),
        E - 1).astype(jnp.int32)
    real_end = poff + counts
    tile_active = (tile_start < real_end[tile_expert]).astype(jnp.int32)
    posm = jnp.where(cnt > 0, pos, PADMAX - 1)
    il = jnp.argmax(cnt, axis=1).astype(jnp.int32)
    ih = (E - 1) - jnp.argmax(cnt[:, ::-1], axis=1).astype(jnp.int32)
    order = jnp.stack([il, ih], axis=1)
    pos01 = jnp.take_along_axis(posm, order, axis=1).astype(jnp.int32)
    w01 = jnp.take_along_axis(wfull, order, axis=1)
    poscat = jnp.concatenate([pos01[:, 0], pos01[:, 1]])
    w0b = jnp.broadcast_to(w01[:, 0:1], (T, 128))
    w1b = jnp.broadcast_to(w01[:, 1:2], (T, 128))

    x_sorted = _make_sc_dispatch()(h2, poscat)
    y_sorted = _moe_call(x_sorted, tile_expert, tile_active,
                         W_gateup, W_down)
    g01 = _sc_gather_rows(y_sorted, poscat)
    return _combine_call(hs2, g01, w0b, w1b)


# speedup vs baseline: 1.0222x; 1.0222x over previous
import functools

import jax
import jax.numpy as jnp
import numpy as np
from jax import lax
from jax.experimental import pallas as pl
from jax.experimental.pallas import tpu as pltpu
from jax.experimental.pallas import tpu_sc as plsc

T = 2048
H = 1024
NH = 16
NKV = 8
HD = 64
E = 8
TOPK = 2
FF = 512
EPS = 1e-6
THETA = 10000.0

BT = 256
BQ = 256
TILE = 128
PADMAX = TOPK * T + E * TILE
NT = PADMAX // TILE
NEG = -1e30

SC_NC = 2
SC_NS = 16
SC_NW = SC_NC * SC_NS


NQK = (NH + NKV) * HD

_G_np = np.zeros((NQK, NH + NKV), dtype=np.float32)
for _c in range(NQK):
    _G_np[_c, _c // HD] = 1.0


def _qkv_body(x_ref, cosb_ref, sinb_ref, wqkv_ref, ln1_ref, wtile_ref,
              g_ref, gt_ref, q_ref, k_ref, v_ref):
    x = x_ref[...]
    var = jnp.mean(x * x, axis=-1, keepdims=True)
    xn = x * lax.rsqrt(var + EPS) * ln1_ref[...]
    qkv = jnp.dot(xn, wqkv_ref[...], preferred_element_type=jnp.float32)
    qk = qkv[:, :NQK]
    ss = jnp.dot(qk * qk, g_ref[...], preferred_element_type=jnp.float32)
    rn = lax.rsqrt(ss * (1.0 / HD) + EPS)
    rnb = jnp.dot(rn, gt_ref[...], preferred_element_type=jnp.float32)
    qkn = qk * rnb * wtile_ref[...]
    rm = jnp.roll(qkn, -HD // 2, axis=1)
    rp = jnp.roll(qkn, HD // 2, axis=1)
    col = lax.broadcasted_iota(jnp.int32, (BT, NQK), 1)
    first_half = (col % HD) < (HD // 2)
    xswap = jnp.where(first_half, -rm, rp)
    rot = qkn * cosb_ref[...] + xswap * sinb_ref[...]
    q_ref[...] = rot[:, :NH * HD]
    k_ref[...] = rot[:, NH * HD:]
    v_ref[...] = qkv[:, NQK:]


def _qkv_call(h, cosb, sinb, Wqkv, ln1_w, wtile, G, GT):
    nb = T // BT
    return pl.pallas_call(
        _qkv_body,
        grid=(nb,),
        in_specs=[
            pl.BlockSpec((BT, H), lambda i: (i, 0)),
            pl.BlockSpec((BT, NQK), lambda i: (i, 0)),
            pl.BlockSpec((BT, NQK), lambda i: (i, 0)),
            pl.BlockSpec((H, (NH + 2 * NKV) * HD), lambda i: (0, 0)),
            pl.BlockSpec((1, H), lambda i: (0, 0)),
            pl.BlockSpec((1, NQK), lambda i: (0, 0)),
            pl.BlockSpec((NQK, NH + NKV), lambda i: (0, 0)),
            pl.BlockSpec((NH + NKV, NQK), lambda i: (0, 0)),
        ],
        out_specs=[
            pl.BlockSpec((BT, NH * HD), lambda i: (i, 0)),
            pl.BlockSpec((BT, NKV * HD), lambda i: (i, 0)),
            pl.BlockSpec((BT, NKV * HD), lambda i: (i, 0)),
        ],
        out_shape=[
            jax.ShapeDtypeStruct((T, NH * HD), jnp.float32),
            jax.ShapeDtypeStruct((T, NKV * HD), jnp.float32),
            jax.ShapeDtypeStruct((T, NKV * HD), jnp.float32),
        ],
        compiler_params=pltpu.CompilerParams(
            dimension_semantics=("arbitrary",)),
    )(h, cosb, sinb, Wqkv, ln1_w, wtile, G, GT)


def _attn_body(q_ref, k_ref, v_ref, o_ref, *, qb0, kvlen):
    i = pl.program_id(0)
    rep = NH // NKV
    scale = HD ** -0.5
    row = (i + qb0) * BQ + lax.broadcasted_iota(jnp.int32, (BQ, kvlen), 0)
    col = lax.broadcasted_iota(jnp.int32, (BQ, kvlen), 1)
    causal = col <= row
    os = []
    for h in range(NH):
        kh = h // rep
        q = q_ref[:, h * HD:(h + 1) * HD] * scale
        k = k_ref[:, kh * HD:(kh + 1) * HD]
        s = lax.dot_general(q, k, (((1,), (1,)), ((), ())),
                            preferred_element_type=jnp.float32)
        s = jnp.where(causal, s, NEG)
        m = jnp.max(s, axis=-1, keepdims=True)
        p = jnp.exp(s - m)
        l = jnp.sum(p, axis=-1, keepdims=True)
        v = v_ref[:, kh * HD:(kh + 1) * HD]
        os.append(jnp.dot(p, v, preferred_element_type=jnp.float32) / l)
    o_ref[...] = jnp.concatenate(os, axis=-1)


def _attn_call_part(q, k, v, qb0, nqb, kvlen):
    return pl.pallas_call(
        functools.partial(_attn_body, qb0=qb0, kvlen=kvlen),
        grid=(nqb,),
        in_specs=[
            pl.BlockSpec((BQ, NH * HD), lambda i: (i + qb0, 0)),
            pl.BlockSpec((kvlen, NKV * HD), lambda i: (0, 0)),
            pl.BlockSpec((kvlen, NKV * HD), lambda i: (0, 0)),
        ],
        out_specs=pl.BlockSpec((BQ, NH * HD), lambda i: (i, 0)),
        out_shape=jax.ShapeDtypeStruct((nqb * BQ, NH * HD), jnp.float32),
        compiler_params=pltpu.CompilerParams(
            dimension_semantics=("arbitrary",)),
    )(q, k, v)


def _oproj_body(a0_ref, a1_ref, a2_ref, a3_ref, res_ref, wo_ref, ln2_ref,
                wg_ref, hs2_ref, h2_ref, wfull_ref):
    i = pl.program_id(0)
    attn = jnp.where(
        i < 2, a0_ref[...],
        jnp.where(i < 4, a1_ref[...],
                  jnp.where(i < 6, a2_ref[...], a3_ref[...])))
    a = jnp.dot(attn, wo_ref[...],
                preferred_element_type=jnp.float32) + res_ref[...]
    hs2_ref[...] = a
    var = jnp.mean(a * a, axis=-1, keepdims=True)
    h2 = a * lax.rsqrt(var + EPS) * ln2_ref[...]
    h2_ref[...] = h2
    logits = jnp.dot(h2, wg_ref[...], preferred_element_type=jnp.float32)
    m = jnp.max(logits, axis=-1, keepdims=True)
    p = jnp.exp(logits - m)
    p = p / jnp.sum(p, axis=-1, keepdims=True)
    ie = lax.broadcasted_iota(jnp.int32, (BT, E), 1)
    m1 = jnp.max(p, axis=-1, keepdims=True)
    i1 = jnp.min(jnp.where(p == m1, ie, E), axis=-1, keepdims=True)
    p2 = jnp.where(ie == i1, NEG, p)
    m2 = jnp.max(p2, axis=-1, keepdims=True)
    i2 = jnp.min(jnp.where(p2 == m2, ie, E), axis=-1, keepdims=True)
    denom = m1 + m2
    wfull_ref[...] = (jnp.where(ie == i1, m1 / denom, 0.0)
                      + jnp.where(ie == i2, m2 / denom, 0.0))


def _oproj_call(attn_parts, res, Wo, ln2_w, Wg):
    nb = T // BT
    return pl.pallas_call(
        _oproj_body,
        grid=(nb,),
        in_specs=[
            pl.BlockSpec((BT, NH * HD),
                         lambda i, p=p: (jnp.clip(i - 2 * p, 0, 1), 0))
            for p in range(4)
        ] + [
            pl.BlockSpec((BT, H), lambda i: (i, 0)),
            pl.BlockSpec((NH * HD, H), lambda i: (0, 0)),
            pl.BlockSpec((1, H), lambda i: (0, 0)),
            pl.BlockSpec((H, E), lambda i: (0, 0)),
        ],
        out_specs=[
            pl.BlockSpec((BT, H), lambda i: (i, 0)),
            pl.BlockSpec((BT, H), lambda i: (i, 0)),
            pl.BlockSpec((BT, E), lambda i: (i, 0)),
        ],
        out_shape=[
            jax.ShapeDtypeStruct((T, H), jnp.float32),
            jax.ShapeDtypeStruct((T, H), jnp.float32),
            jax.ShapeDtypeStruct((T, E), jnp.float32),
        ],
        compiler_params=pltpu.CompilerParams(
            dimension_semantics=("arbitrary",)),
    )(*attn_parts, res, Wo, ln2_w, Wg)


@functools.lru_cache(maxsize=None)
def _make_sc_gather(n_rows, width):
    n_per = n_rows // SC_NW
    ch = min(32, n_per)
    nch = n_per // ch
    nbuf = min(3, nch)
    mesh = plsc.VectorSubcoreMesh(core_axis_name="c", subcore_axis_name="s")

    @functools.partial(
        pl.kernel, mesh=mesh,
        out_type=jax.ShapeDtypeStruct((n_rows, width), jnp.int32),
        scratch_types=[
            pltpu.VMEM((n_per,), jnp.int32),
        ] + [pltpu.VMEM((ch, width), jnp.int32)] * nbuf
          + [pltpu.SemaphoreType.DMA] * (2 * nbuf),
    )
    def gk(table_hbm, idx_hbm, out_hbm, idx_v, *scr):
        bufs = scr[:nbuf]
        gsems = scr[nbuf:2 * nbuf]
        osems = scr[2 * nbuf:]
        wid = lax.axis_index("s") * SC_NC + lax.axis_index("c")
        base = wid * n_per
        pltpu.sync_copy(idx_hbm.at[pl.ds(base, n_per)], idx_v)

        def start_gather(c):
            b = c % nbuf
            return pltpu.async_copy(
                table_hbm.at[idx_v.at[pl.ds(c * ch, ch)]], bufs[b], gsems[b])

        g = [None] * nch
        o = [None] * nch
        for c in range(nbuf):
            g[c] = start_gather(c)
        for c in range(nch):
            g[c].wait()
            o[c] = pltpu.async_copy(
                bufs[c % nbuf], out_hbm.at[pl.ds(base + c * ch, ch)],
                osems[c % nbuf])
            if c + nbuf < nch:
                o[c].wait()
                g[c + nbuf] = start_gather(c + nbuf)
        for c in range(max(0, nch - nbuf), nch):
            o[c].wait()

    return gk


@functools.lru_cache(maxsize=None)
def _make_sc_dispatch():
    n_per = T // SC_NW
    ch = min(32, n_per)
    nch = n_per // ch
    mesh = plsc.VectorSubcoreMesh(core_axis_name="c", subcore_axis_name="s")

    @functools.partial(
        pl.kernel, mesh=mesh,
        out_type=jax.ShapeDtypeStruct((PADMAX, H), jnp.float32),
        scratch_types=[
            pltpu.VMEM((n_per,), jnp.int32),
            pltpu.VMEM((n_per,), jnp.int32),
        ] + [pltpu.VMEM((ch, H), jnp.float32)] * nch
          + [pltpu.SemaphoreType.DMA] * (3 * nch),
    )
    def sk(src_hbm, idx_hbm, out_hbm, idx0_v, idx1_v, *scr):
        bufs = scr[:nch]
        lsems = scr[nch:2 * nch]
        ssems = scr[2 * nch:]
        wid = lax.axis_index("s") * SC_NC + lax.axis_index("c")
        base = wid * n_per
        pltpu.sync_copy(idx_hbm.at[pl.ds(base, n_per)], idx0_v)
        pltpu.sync_copy(idx_hbm.at[pl.ds(T + base, n_per)], idx1_v)
        loads = [pltpu.async_copy(src_hbm.at[pl.ds(base + c * ch, ch)],
                                  bufs[c], lsems[c]) for c in range(nch)]
        outs = []
        for c in range(nch):
            loads[c].wait()
            outs.append(pltpu.async_copy(
                bufs[c], out_hbm.at[idx0_v.at[pl.ds(c * ch, ch)]],
                ssems[2 * c]))
            outs.append(pltpu.async_copy(
                bufs[c], out_hbm.at[idx1_v.at[pl.ds(c * ch, ch)]],
                ssems[2 * c + 1]))
        for o in outs:
            o.wait()

    return sk


def _sc_gather_rows(table, idx):
    dt = table.dtype
    if dt.itemsize == 2:
        ti = lax.bitcast_convert_type(
            table.reshape(table.shape[0], -1, 2), jnp.int32)
    else:
        ti = lax.bitcast_convert_type(table, jnp.int32)
    out = _make_sc_gather(idx.shape[0], ti.shape[1])(ti, idx)
    res = lax.bitcast_convert_type(out, dt)
    return res.reshape(idx.shape[0], H)


def _moe_body(te_ref, x_ref, wgu_ref, wd_ref, y_ref):
    x = x_ref[...]
    gu = jnp.dot(x, wgu_ref[0], preferred_element_type=jnp.float32)
    g = gu[:, :FF]
    u = gu[:, FF:]
    act = g * jax.nn.sigmoid(g) * u
    y_ref[...] = jnp.dot(act, wd_ref[0], preferred_element_type=jnp.float32)


def _moe_call(x_sorted, tile_expert, W_gateup, W_down):
    grid_spec = pltpu.PrefetchScalarGridSpec(
        num_scalar_prefetch=1,
        grid=(NT,),
        in_specs=[
            pl.BlockSpec((TILE, H), lambda g, te: (g, 0)),
            pl.BlockSpec((1, H, 2 * FF), lambda g, te: (te[g], 0, 0)),
            pl.BlockSpec((1, FF, H), lambda g, te: (te[g], 0, 0)),
        ],
        out_specs=pl.BlockSpec((TILE, H), lambda g, te: (g, 0)),
    )
    return pl.pallas_call(
        _moe_body,
        grid_spec=grid_spec,
        out_shape=jax.ShapeDtypeStruct((PADMAX, H), jnp.float32),
        compiler_params=pltpu.CompilerParams(
            dimension_semantics=("arbitrary",)),
    )(tile_expert, x_sorted, W_gateup, W_down)


def _combine_body(res_ref, g0_ref, g1_ref, w0_ref, w1_ref, out_ref):
    out_ref[...] = (res_ref[...]
                    + g0_ref[...].astype(jnp.float32) * w0_ref[:, :1]
                    + g1_ref[...].astype(jnp.float32) * w1_ref[:, :1])


def _combine_call(res, g01, w0b, w1b):
    nb = T // BT
    return pl.pallas_call(
        _combine_body,
        grid=(nb,),
        in_specs=[
            pl.BlockSpec((BT, H), lambda i: (i, 0)),
            pl.BlockSpec((BT, H), lambda i: (i, 0)),
            pl.BlockSpec((BT, H), lambda i: (i + T // BT, 0)),
            pl.BlockSpec((BT, 128), lambda i: (i, 0)),
            pl.BlockSpec((BT, 128), lambda i: (i, 0)),
        ],
        out_specs=pl.BlockSpec((BT, H), lambda i: (i, 0)),
        out_shape=jax.ShapeDtypeStruct((T, H), jnp.float32),
        compiler_params=pltpu.CompilerParams(
            dimension_semantics=("arbitrary",)),
    )(res, g01, g01, w0b, w1b)


def kernel(hidden_states, positions, Wqkv, Wo, q_norm_w, k_norm_w,
           ln1_w, ln2_w, Wg, W_gateup, W_down):
    f32 = jnp.float32
    inv_freq = 1.0 / (THETA ** (np.arange(0, HD, 2, dtype=np.float32) / HD))
    freqs = positions.astype(f32)[:, None] * inv_freq[None, :]
    cos64 = jnp.cos(jnp.concatenate([freqs, freqs], axis=1))
    sin64 = jnp.sin(jnp.concatenate([freqs, freqs], axis=1))
    cosb = jnp.tile(cos64, (1, NH + NKV))
    sinb = jnp.tile(sin64, (1, NH + NKV))
    wtile = jnp.concatenate(
        [jnp.tile(q_norm_w, NH), jnp.tile(k_norm_w, NKV)]).reshape(1, NQK)
    G = jnp.asarray(_G_np)

    q, k, v = _qkv_call(hidden_states, cosb, sinb, Wqkv,
                        ln1_w.reshape(1, H), wtile, G, G.T)
    attn_parts = [
        _attn_call_part(q, k, v, 2 * p, 2, (p + 1) * (T // 4))
        for p in range(4)
    ]
    hs2, h2, wfull = _oproj_call(attn_parts, hidden_states, Wo,
                                 ln2_w.reshape(1, H), Wg)

    cnt = (wfull > 0.0).astype(jnp.int32)
    csum = jnp.cumsum(cnt, axis=0)
    prefix = csum - cnt
    counts = csum[-1]
    pcounts = ((counts + TILE - 1) // TILE) * TILE
    pend = jnp.cumsum(pcounts)
    poff = pend - pcounts
    pos = poff[None, :] + prefix
    tile_expert = jnp.minimum(
        jnp.searchsorted(pend, jnp.arange(NT, dtype=jnp.int32) * TILE,
                         side="right"),
        E - 1).astype(jnp.int32)
    posm = jnp.where(cnt > 0, pos, PADMAX - 1)
    il = jnp.argmax(cnt, axis=1).astype(jnp.int32)
    ih = (E - 1) - jnp.argmax(cnt[:, ::-1], axis=1).astype(jnp.int32)
    order = jnp.stack([il, ih], axis=1)
    pos01 = jnp.take_along_axis(posm, order, axis=1).astype(jnp.int32)
    w01 = jnp.take_along_axis(wfull, order, axis=1)
    poscat = jnp.concatenate([pos01[:, 0], pos01[:, 1]])
    w0b = jnp.broadcast_to(w01[:, 0:1], (T, 128))
    w1b = jnp.broadcast_to(w01[:, 1:2], (T, 128))

    x_sorted = _make_sc_dispatch()(h2, poscat)
    y_sorted = _moe_call(x_sorted, tile_expert, W_gateup, W_down)
    g01 = _sc_gather_rows(y_sorted, poscat)
    return _combine_call(hs2, g01, w0b, w1b)
